# Initial kernel scaffold; baseline (speedup 1.0000x reference)
import jax
import jax.numpy as jnp
from jax.experimental import pallas as pl


def kernel(x, edge_index, W1l, b1l, W1r, W2l, b2l, W2r):
    raise NotImplementedError("write your pallas kernel here")


# speedup vs baseline: 8.2154x; 8.2154x over previous
import functools

import jax
import jax.numpy as jnp
from jax import lax
from jax.experimental import pallas as pl
from jax.experimental.pallas import tpu as pltpu
from jax.experimental.pallas import tpu_sc as plsc

N = 50000
E = 800000
H = 64
NC, NS = 2, 16
EPAD = 802816
ROWS = EPAD // 128
RPT = ROWS // NS
RPC = ROWS // NC
RPCT = RPC // NS
NPAD = 51200
APT = NPAD // NS
BM = 2048
GRID = NPAD // BM
ZR = 400

_mesh = plsc.VectorSubcoreMesh(core_axis_name="c", subcore_axis_name="s")
_sc_params = pltpu.CompilerParams(use_tc_tiling_on_sc=False)


def _mm1_body(x_ref, wl_ref, wr_ref, y4_ref, z_ref):
    xb = x_ref[...]
    y = lax.dot_general(xb, wl_ref[...], (((1,), (1,)), ((), ())),
                        preferred_element_type=jnp.float32)
    for q in range(4):
        y4_ref[q] = y[:, 16 * q:16 * (q + 1)]
    z_ref[...] = lax.dot_general(xb, wr_ref[...], (((1,), (1,)), ((), ())),
                                 preferred_element_type=jnp.float32)


def _mid_body(agg_ref, cnt_ref, z_ref, b1_ref, w2l_ref, w2r_ref, b2_ref,
              h2_ref, hb_ref):
    a = jnp.concatenate([agg_ref[q] for q in range(4)], axis=1)
    cnt = cnt_ref[0] + cnt_ref[1]
    d = 1.0 / jnp.maximum(cnt, 1.0)
    h = jnp.maximum(a * d[:, None] + b1_ref[0][None, :] + z_ref[...], 0.0)
    h2_ref[...] = lax.dot_general(h, w2l_ref[...], (((1,), (1,)), ((), ())),
                                  preferred_element_type=jnp.float32)
    hb_ref[...] = lax.dot_general(h, w2r_ref[...], (((1,), (1,)), ((), ())),
                                  preferred_element_type=jnp.float32) \
        + b2_ref[0][None, :]


def _fin_body(agg2_ref, cnt_ref, hb_ref, out_ref):
    a = agg2_ref[0] + agg2_ref[1]
    cnt = cnt_ref[0] + cnt_ref[1]
    d = 1.0 / jnp.maximum(cnt, 1.0)
    out_ref[...] = a * d[:, None] + hb_ref[...]


def _fill_zero(zb_v):
    def body(r, carry):
        zb_v[r, :] = jnp.zeros((16,), jnp.float32)
        return carry
    lax.fori_loop(0, ZR, body, 0)


def _zero_shared_stripe(zb_v, buf_sh, s):
    for k in range(APT // ZR):
        pltpu.sync_copy(zb_v, buf_sh.at[pl.ds(s * APT + k * ZR, ZR)])


def _edge_loop(n_iters, base_fn, tab_sh, acc_sh, src2d, dst2d,
               src_v, dst_v, rows_v, gsem):
    def body(g, carry):
        base = base_fn(g)
        pltpu.sync_copy(src2d.at[pl.ds(base, 4)], src_v)
        pltpu.sync_copy(dst2d.at[pl.ds(base, 4)], dst_v)
        cps = [pltpu.async_copy(tab_sh.at[src_v.at[j]],
                                rows_v.at[pl.ds(j * 128, 128)], gsem)
               for j in range(4)]
        for cp in cps:
            cp.wait()
        for j in range(4):
            pltpu.sync_copy(rows_v.at[pl.ds(j * 128, 128)],
                            acc_sh.at[dst_v.at[j]], add=True)
        return carry
    lax.fori_loop(0, n_iters, body, 0)


@functools.partial(
    pl.kernel,
    out_type=[jax.ShapeDtypeStruct((4, NPAD, 16), jnp.float32),
              jax.ShapeDtypeStruct((NC * NPAD,), jnp.float32)],
    mesh=_mesh,
    scratch_types=[
        pltpu.VMEM_SHARED((NPAD, 16), jnp.float32),
        pltpu.VMEM_SHARED((NPAD, 16), jnp.float32),
        pltpu.VMEM_SHARED((NPAD,), jnp.float32),
        pltpu.VMEM((ZR, 16), jnp.float32),
        pltpu.VMEM((APT,), jnp.float32),
        pltpu.VMEM((4, 128), jnp.int32),
        pltpu.VMEM((4, 128), jnp.int32),
        pltpu.VMEM((512, 16), jnp.float32),
        pltpu.VMEM((128,), jnp.float32),
        pltpu.SemaphoreType.DMA,
    ],
    compiler_params=_sc_params,
)
def _agg1_kernel(y4, src2d, dst2d, agg_out, cnt_out,
                 tab_sh, acc_sh, cnt_sh, zb_v, zb1_v,
                 src_v, dst_v, rows_v, ones_v, gsem):
    c = lax.axis_index("c")
    s = lax.axis_index("s")
    for k in range(8):
        ones_v[pl.ds(k * 16, 16)] = jnp.full((16,), 1.0, jnp.float32)
    _fill_zero(zb_v)

    def zb1_body(r, carry):
        zb1_v[pl.ds(r * 16, 16)] = jnp.zeros((16,), jnp.float32)
        return carry
    lax.fori_loop(0, APT // 16, zb1_body, 0)
    pltpu.sync_copy(zb1_v, cnt_sh.at[pl.ds(s * APT, APT)])

    for q in range(2):
        qi = c * 2 + q
        pltpu.sync_copy(y4.at[qi, pl.ds(s * APT, APT)],
                        tab_sh.at[pl.ds(s * APT, APT)])
        _zero_shared_stripe(zb_v, acc_sh, s)
        plsc.subcore_barrier()

        _edge_loop(RPT // 4, lambda g: s * RPT + g * 4,
                   tab_sh, acc_sh, src2d, dst2d, src_v, dst_v, rows_v, gsem)

        if q == 0:
            def cnt_body(g, carry):
                base = c * RPC + s * RPCT + g * 4
                pltpu.sync_copy(dst2d.at[pl.ds(base, 4)], dst_v)
                for j in range(4):
                    pltpu.sync_copy(ones_v, cnt_sh.at[dst_v.at[j]], add=True)
                return carry
            lax.fori_loop(0, RPCT // 4, cnt_body, 0)

        plsc.subcore_barrier()
        pltpu.sync_copy(acc_sh.at[pl.ds(s * APT, APT)],
                        agg_out.at[qi, pl.ds(s * APT, APT)])
        if q == 0:
            pltpu.sync_copy(cnt_sh.at[pl.ds(s * APT, APT)],
                            cnt_out.at[pl.ds(c * NPAD + s * APT, APT)])
        plsc.subcore_barrier()


@functools.partial(
    pl.kernel,
    out_type=jax.ShapeDtypeStruct((NC, NPAD, 16), jnp.float32),
    mesh=_mesh,
    scratch_types=[
        pltpu.VMEM_SHARED((NPAD, 16), jnp.float32),
        pltpu.VMEM_SHARED((NPAD, 16), jnp.float32),
        pltpu.VMEM((ZR, 16), jnp.float32),
        pltpu.VMEM((4, 128), jnp.int32),
        pltpu.VMEM((4, 128), jnp.int32),
        pltpu.VMEM((512, 16), jnp.float32),
        pltpu.SemaphoreType.DMA,
    ],
    compiler_params=_sc_params,
)
def _agg2_kernel(h2pad, src2d, dst2d, agg_out,
                 tab_sh, acc_sh, zb_v, src_v, dst_v, rows_v, gsem):
    c = lax.axis_index("c")
    s = lax.axis_index("s")
    pltpu.sync_copy(h2pad.at[pl.ds(s * APT, APT)],
                    tab_sh.at[pl.ds(s * APT, APT)])
    _fill_zero(zb_v)
    _zero_shared_stripe(zb_v, acc_sh, s)
    plsc.subcore_barrier()

    _edge_loop(RPCT // 4, lambda g: c * RPC + s * RPCT + g * 4,
               tab_sh, acc_sh, src2d, dst2d, src_v, dst_v, rows_v, gsem)

    plsc.subcore_barrier()
    pltpu.sync_copy(acc_sh.at[pl.ds(s * APT, APT)],
                    agg_out.at[c, pl.ds(s * APT, APT)])


def kernel(x, edge_index, W1l, b1l, W1r, W2l, b2l, W2r):
    src = edge_index[0].astype(jnp.int32)
    dst = edge_index[1].astype(jnp.int32)
    pad = EPAD - E
    psrc = (jnp.arange(pad, dtype=jnp.int32) * 7919) % N
    pdst = N + jnp.arange(pad, dtype=jnp.int32) % (NPAD - N)
    src2d = jnp.concatenate([src, psrc]).reshape(ROWS, 128)
    dst2d = jnp.concatenate([dst, pdst]).reshape(ROWS, 128)

    y4, z = pl.pallas_call(
        _mm1_body,
        grid=(GRID,),
        in_specs=[pl.BlockSpec((BM, H), lambda i: (i, 0)),
                  pl.BlockSpec((H, H), lambda i: (0, 0)),
                  pl.BlockSpec((H, H), lambda i: (0, 0))],
        out_specs=[pl.BlockSpec((4, BM, 16), lambda i: (0, i, 0)),
                   pl.BlockSpec((BM, H), lambda i: (i, 0))],
        out_shape=[jax.ShapeDtypeStruct((4, NPAD, 16), jnp.float32),
                   jax.ShapeDtypeStruct((NPAD, H), jnp.float32)],
    )(x, W1l, W1r)

    agg1, cnt_flat = _agg1_kernel(y4, src2d, dst2d)
    cnt = cnt_flat.reshape(NC, NPAD)

    W2lp = jnp.zeros((16, H), jnp.float32).at[:3].set(W2l)
    W2rp = jnp.zeros((16, H), jnp.float32).at[:3].set(W2r)
    b2p = jnp.zeros((1, 16), jnp.float32).at[0, :3].set(b2l)
    h2pad, hb = pl.pallas_call(
        _mid_body,
        grid=(GRID,),
        in_specs=[pl.BlockSpec((4, BM, 16), lambda i: (0, i, 0)),
                  pl.BlockSpec((2, BM), lambda i: (0, i)),
                  pl.BlockSpec((BM, H), lambda i: (i, 0)),
                  pl.BlockSpec((1, H), lambda i: (0, 0)),
                  pl.BlockSpec((16, H), lambda i: (0, 0)),
                  pl.BlockSpec((16, H), lambda i: (0, 0)),
                  pl.BlockSpec((1, 16), lambda i: (0, 0))],
        out_specs=[pl.BlockSpec((BM, 16), lambda i: (i, 0)),
                   pl.BlockSpec((BM, 16), lambda i: (i, 0))],
        out_shape=[jax.ShapeDtypeStruct((NPAD, 16), jnp.float32),
                   jax.ShapeDtypeStruct((NPAD, 16), jnp.float32)],
    )(agg1, cnt, z, b1l.reshape(1, H), W2lp, W2rp, b2p)

    agg2 = _agg2_kernel(h2pad, src2d, dst2d)

    out16 = pl.pallas_call(
        _fin_body,
        grid=(GRID,),
        in_specs=[pl.BlockSpec((2, BM, 16), lambda i: (0, i, 0)),
                  pl.BlockSpec((2, BM), lambda i: (0, i)),
                  pl.BlockSpec((BM, 16), lambda i: (i, 0))],
        out_specs=pl.BlockSpec((BM, 16), lambda i: (i, 0)),
        out_shape=jax.ShapeDtypeStruct((NPAD, 16), jnp.float32),
    )(agg2, cnt, hb)
    return out16[:N, :3]


# speedup vs baseline: 10.2612x; 1.2490x over previous
import functools

import jax
import jax.numpy as jnp
from jax import lax
from jax.experimental import pallas as pl
from jax.experimental.pallas import tpu as pltpu
from jax.experimental.pallas import tpu_sc as plsc

N = 50000
E = 800000
H = 64
NC, NS = 2, 16
EPAD = 802816
ROWS = EPAD // 128
RPT = ROWS // NS
RPC = ROWS // NC
RPCT = RPC // NS
NPAD = 51200
APT = NPAD // NS
BM = 2048
GRID = NPAD // BM
ZR = 400

_mesh = plsc.VectorSubcoreMesh(core_axis_name="c", subcore_axis_name="s")
_sc_params = pltpu.CompilerParams(use_tc_tiling_on_sc=False)


def _mm1_body(x_ref, wl_ref, wr_ref, y4_ref, z_ref):
    xb = x_ref[...]
    y = lax.dot_general(xb, wl_ref[...], (((1,), (1,)), ((), ())),
                        preferred_element_type=jnp.float32)
    for q in range(4):
        y4_ref[q] = y[:, 16 * q:16 * (q + 1)]
    z_ref[...] = lax.dot_general(xb, wr_ref[...], (((1,), (1,)), ((), ())),
                                 preferred_element_type=jnp.float32)


def _mid_body(agg_ref, cnt_ref, z_ref, b1_ref, w2l_ref, w2r_ref, b2_ref,
              h2_ref, hb_ref):
    a = jnp.concatenate([agg_ref[q] for q in range(4)], axis=1)
    cnt = cnt_ref[0] + cnt_ref[1]
    d = 1.0 / jnp.maximum(cnt, 1.0)
    h = jnp.maximum(a * d[:, None] + b1_ref[0][None, :] + z_ref[...], 0.0)
    h2_ref[...] = lax.dot_general(h, w2l_ref[...], (((1,), (1,)), ((), ())),
                                  preferred_element_type=jnp.float32)
    hb_ref[...] = lax.dot_general(h, w2r_ref[...], (((1,), (1,)), ((), ())),
                                  preferred_element_type=jnp.float32) \
        + b2_ref[0][None, :]


def _fin_body(agg2_ref, cnt_ref, hb_ref, out_ref):
    a = agg2_ref[0] + agg2_ref[1]
    cnt = cnt_ref[0] + cnt_ref[1]
    d = 1.0 / jnp.maximum(cnt, 1.0)
    out_ref[...] = a * d[:, None] + hb_ref[...]


def _fill_zero(zb_v):
    def body(r, carry):
        zb_v[r, :] = jnp.zeros((16,), jnp.float32)
        return carry
    lax.fori_loop(0, ZR, body, 0)


def _zero_shared_stripe(zb_v, buf_sh, s):
    for k in range(APT // ZR):
        pltpu.sync_copy(zb_v, buf_sh.at[pl.ds(s * APT + k * ZR, ZR)])


def _pipe4(n_groups, fire, drain, process, bufa, bufb):
    npairs, tail = n_groups // 2, n_groups % 2
    fire(0, bufa)

    def body(p, carry):
        g = p * 2
        fire(g + 1, bufb)
        drain(bufa)
        process(bufa)
        fire(jnp.minimum(g + 2, n_groups - 1), bufa)
        drain(bufb)
        process(bufb)
        return carry
    lax.fori_loop(0, npairs, body, 0)
    drain(bufa)
    if tail:
        process(bufa)


def _edge_loop(n_groups, base, src2d, dst2d, tab_sh, acc_sh,
               sa_v, da_v, sb_v, db_v, rows_v, gsem, isem):
    def fire(g, buf):
        s_v, d_v = buf
        pltpu.async_copy(src2d.at[pl.ds(base + g * 4, 4)], s_v, isem)
        pltpu.async_copy(dst2d.at[pl.ds(base + g * 4, 4)], d_v, isem)

    def drain(buf):
        s_v, d_v = buf
        pltpu.make_async_copy(src2d.at[pl.ds(base, 4)], s_v, isem).wait()
        pltpu.make_async_copy(dst2d.at[pl.ds(base, 4)], d_v, isem).wait()

    def process(buf):
        s_v, d_v = buf
        cps = [pltpu.async_copy(tab_sh.at[s_v.at[j]],
                                rows_v.at[pl.ds(j * 128, 128)], gsem)
               for j in range(4)]
        for cp in cps:
            cp.wait()
        for j in range(4):
            pltpu.sync_copy(rows_v.at[pl.ds(j * 128, 128)],
                            acc_sh.at[d_v.at[j]], add=True)

    _pipe4(n_groups, fire, drain, process, (sa_v, da_v), (sb_v, db_v))


@functools.partial(
    pl.kernel,
    out_type=[jax.ShapeDtypeStruct((4, NPAD, 16), jnp.float32),
              jax.ShapeDtypeStruct((NC * NPAD,), jnp.float32)],
    mesh=_mesh,
    scratch_types=[
        pltpu.VMEM_SHARED((NPAD, 16), jnp.float32),
        pltpu.VMEM_SHARED((NPAD, 16), jnp.float32),
        pltpu.VMEM_SHARED((NPAD,), jnp.float32),
        pltpu.VMEM((ZR, 16), jnp.float32),
        pltpu.VMEM((APT,), jnp.float32),
        pltpu.VMEM((4, 128), jnp.int32),
        pltpu.VMEM((4, 128), jnp.int32),
        pltpu.VMEM((4, 128), jnp.int32),
        pltpu.VMEM((4, 128), jnp.int32),
        pltpu.VMEM((512, 16), jnp.float32),
        pltpu.VMEM((128,), jnp.float32),
        pltpu.SemaphoreType.DMA,
        pltpu.SemaphoreType.DMA,
    ],
    compiler_params=_sc_params,
)
def _agg1_kernel(y4, src2d, dst2d, agg_out, cnt_out,
                 tab_sh, acc_sh, cnt_sh, zb_v, zb1_v,
                 sa_v, da_v, sb_v, db_v, rows_v, ones_v, gsem, isem):
    c = lax.axis_index("c")
    s = lax.axis_index("s")
    for k in range(8):
        ones_v[pl.ds(k * 16, 16)] = jnp.full((16,), 1.0, jnp.float32)
    _fill_zero(zb_v)

    def zb1_body(r, carry):
        zb1_v[pl.ds(r * 16, 16)] = jnp.zeros((16,), jnp.float32)
        return carry
    lax.fori_loop(0, APT // 16, zb1_body, 0)
    pltpu.sync_copy(zb1_v, cnt_sh.at[pl.ds(s * APT, APT)])

    for q in range(2):
        qi = c * 2 + q
        pltpu.sync_copy(y4.at[qi, pl.ds(s * APT, APT)],
                        tab_sh.at[pl.ds(s * APT, APT)])
        _zero_shared_stripe(zb_v, acc_sh, s)
        plsc.subcore_barrier()

        _edge_loop(RPT // 4, s * RPT, src2d, dst2d, tab_sh, acc_sh,
                   sa_v, da_v, sb_v, db_v, rows_v, gsem, isem)

        if q == 0:
            cbase = c * RPC + s * RPCT

            def cfire(g, d_v):
                pltpu.async_copy(dst2d.at[pl.ds(cbase + g * 4, 4)],
                                 d_v, isem)

            def cdrain(d_v):
                pltpu.make_async_copy(dst2d.at[pl.ds(cbase, 4)],
                                      d_v, isem).wait()

            def cproc(d_v):
                for j in range(4):
                    pltpu.sync_copy(ones_v, cnt_sh.at[d_v.at[j]], add=True)

            _pipe4(RPCT // 4, cfire, cdrain, cproc, da_v, db_v)

        plsc.subcore_barrier()
        pltpu.sync_copy(acc_sh.at[pl.ds(s * APT, APT)],
                        agg_out.at[qi, pl.ds(s * APT, APT)])
        if q == 0:
            pltpu.sync_copy(cnt_sh.at[pl.ds(s * APT, APT)],
                            cnt_out.at[pl.ds(c * NPAD + s * APT, APT)])
        plsc.subcore_barrier()


@functools.partial(
    pl.kernel,
    out_type=jax.ShapeDtypeStruct((NC, NPAD, 16), jnp.float32),
    mesh=_mesh,
    scratch_types=[
        pltpu.VMEM_SHARED((NPAD, 16), jnp.float32),
        pltpu.VMEM_SHARED((NPAD, 16), jnp.float32),
        pltpu.VMEM((ZR, 16), jnp.float32),
        pltpu.VMEM((4, 128), jnp.int32),
        pltpu.VMEM((4, 128), jnp.int32),
        pltpu.VMEM((4, 128), jnp.int32),
        pltpu.VMEM((4, 128), jnp.int32),
        pltpu.VMEM((512, 16), jnp.float32),
        pltpu.SemaphoreType.DMA,
        pltpu.SemaphoreType.DMA,
    ],
    compiler_params=_sc_params,
)
def _agg2_kernel(h2pad, src2d, dst2d, agg_out,
                 tab_sh, acc_sh, zb_v, sa_v, da_v, sb_v, db_v,
                 rows_v, gsem, isem):
    c = lax.axis_index("c")
    s = lax.axis_index("s")
    pltpu.sync_copy(h2pad.at[pl.ds(s * APT, APT)],
                    tab_sh.at[pl.ds(s * APT, APT)])
    _fill_zero(zb_v)
    _zero_shared_stripe(zb_v, acc_sh, s)
    plsc.subcore_barrier()

    _edge_loop(RPCT // 4, c * RPC + s * RPCT, src2d, dst2d, tab_sh, acc_sh,
               sa_v, da_v, sb_v, db_v, rows_v, gsem, isem)

    plsc.subcore_barrier()
    pltpu.sync_copy(acc_sh.at[pl.ds(s * APT, APT)],
                    agg_out.at[c, pl.ds(s * APT, APT)])


def kernel(x, edge_index, W1l, b1l, W1r, W2l, b2l, W2r):
    src = edge_index[0].astype(jnp.int32)
    dst = edge_index[1].astype(jnp.int32)
    pad = EPAD - E
    psrc = (jnp.arange(pad, dtype=jnp.int32) * 7919) % N
    pdst = N + jnp.arange(pad, dtype=jnp.int32) % (NPAD - N)
    src2d = jnp.concatenate([src, psrc]).reshape(ROWS, 128)
    dst2d = jnp.concatenate([dst, pdst]).reshape(ROWS, 128)

    y4, z = pl.pallas_call(
        _mm1_body,
        grid=(GRID,),
        in_specs=[pl.BlockSpec((BM, H), lambda i: (i, 0)),
                  pl.BlockSpec((H, H), lambda i: (0, 0)),
                  pl.BlockSpec((H, H), lambda i: (0, 0))],
        out_specs=[pl.BlockSpec((4, BM, 16), lambda i: (0, i, 0)),
                   pl.BlockSpec((BM, H), lambda i: (i, 0))],
        out_shape=[jax.ShapeDtypeStruct((4, NPAD, 16), jnp.float32),
                   jax.ShapeDtypeStruct((NPAD, H), jnp.float32)],
    )(x, W1l, W1r)

    agg1, cnt_flat = _agg1_kernel(y4, src2d, dst2d)
    cnt = cnt_flat.reshape(NC, NPAD)

    W2lp = jnp.zeros((16, H), jnp.float32).at[:3].set(W2l)
    W2rp = jnp.zeros((16, H), jnp.float32).at[:3].set(W2r)
    b2p = jnp.zeros((1, 16), jnp.float32).at[0, :3].set(b2l)
    h2pad, hb = pl.pallas_call(
        _mid_body,
        grid=(GRID,),
        in_specs=[pl.BlockSpec((4, BM, 16), lambda i: (0, i, 0)),
                  pl.BlockSpec((2, BM), lambda i: (0, i)),
                  pl.BlockSpec((BM, H), lambda i: (i, 0)),
                  pl.BlockSpec((1, H), lambda i: (0, 0)),
                  pl.BlockSpec((16, H), lambda i: (0, 0)),
                  pl.BlockSpec((16, H), lambda i: (0, 0)),
                  pl.BlockSpec((1, 16), lambda i: (0, 0))],
        out_specs=[pl.BlockSpec((BM, 16), lambda i: (i, 0)),
                   pl.BlockSpec((BM, 16), lambda i: (i, 0))],
        out_shape=[jax.ShapeDtypeStruct((NPAD, 16), jnp.float32),
                   jax.ShapeDtypeStruct((NPAD, 16), jnp.float32)],
    )(agg1, cnt, z, b1l.reshape(1, H), W2lp, W2rp, b2p)

    agg2 = _agg2_kernel(h2pad, src2d, dst2d)

    out16 = pl.pallas_call(
        _fin_body,
        grid=(GRID,),
        in_specs=[pl.BlockSpec((2, BM, 16), lambda i: (0, i, 0)),
                  pl.BlockSpec((2, BM), lambda i: (0, i)),
                  pl.BlockSpec((BM, 16), lambda i: (i, 0))],
        out_specs=pl.BlockSpec((BM, 16), lambda i: (i, 0)),
        out_shape=jax.ShapeDtypeStruct((NPAD, 16), jnp.float32),
    )(agg2, cnt, hb)
    return out16[:N, :3]


# speedup vs baseline: 11.1465x; 1.0863x over previous
import functools

import jax
import jax.numpy as jnp
from jax import lax
from jax.experimental import pallas as pl
from jax.experimental.pallas import tpu as pltpu
from jax.experimental.pallas import tpu_sc as plsc

N = 50000
E = 800000
H = 64
NC, NS = 2, 16
EPAD = 802816
ROWS = EPAD // 128
RPT = ROWS // NS
RPC = ROWS // NC
RPCT = RPC // NS
NPAD = 51200
APT = NPAD // NS
BM = 2048
GRID = NPAD // BM
ZR = 400

_mesh = plsc.VectorSubcoreMesh(core_axis_name="c", subcore_axis_name="s")
_sc_params = pltpu.CompilerParams(use_tc_tiling_on_sc=False)


def _mm1_body(x_ref, wl_ref, wr_ref, y1_ref, z_ref):
    xb = x_ref[...]
    y1_ref[...] = lax.dot_general(xb, wl_ref[...], (((1,), (1,)), ((), ())),
                                  preferred_element_type=jnp.float32)
    z_ref[...] = lax.dot_general(xb, wr_ref[...], (((1,), (1,)), ((), ())),
                                 preferred_element_type=jnp.float32)


def _mid_body(agg_ref, cnt_ref, z_ref, b1_ref, w2l_ref, w2r_ref, b2_ref,
              h2_ref, hb_ref):
    cnt = cnt_ref[0] + cnt_ref[1]
    d = 1.0 / jnp.maximum(cnt, 1.0)
    h = jnp.maximum(agg_ref[...] * d[:, None] + b1_ref[0][None, :]
                    + z_ref[...], 0.0)
    h2_ref[...] = lax.dot_general(h, w2l_ref[...], (((1,), (1,)), ((), ())),
                                  preferred_element_type=jnp.float32)
    hb_ref[...] = lax.dot_general(h, w2r_ref[...], (((1,), (1,)), ((), ())),
                                  preferred_element_type=jnp.float32) \
        + b2_ref[0][None, :]


def _fin_body(agg2_ref, cnt_ref, hb_ref, out_ref):
    a = agg2_ref[:, :16] + agg2_ref[:, 16:]
    cnt = cnt_ref[0] + cnt_ref[1]
    d = 1.0 / jnp.maximum(cnt, 1.0)
    out_ref[...] = a * d[:, None] + hb_ref[...]


def _fill_zero(zb_v):
    def body(r, carry):
        zb_v[r, :] = jnp.zeros((16,), jnp.float32)
        return carry
    lax.fori_loop(0, ZR, body, 0)


def _zero_shared_stripe(zb_v, buf_sh, s):
    for k in range(APT // ZR):
        pltpu.sync_copy(zb_v, buf_sh.at[pl.ds(s * APT + k * ZR, ZR)])


def _pipe4(n_groups, fire, drain, process, bufa, bufb):
    npairs, tail = n_groups // 2, n_groups % 2
    fire(0, bufa)

    def body(p, carry):
        g = p * 2
        fire(g + 1, bufb)
        drain(bufa)
        process(bufa)
        fire(jnp.minimum(g + 2, n_groups - 1), bufa)
        drain(bufb)
        process(bufb)
        return carry
    lax.fori_loop(0, npairs, body, 0)
    drain(bufa)
    if tail:
        process(bufa)


def _edge_loop(n_groups, base, src2d, dst2d, tab_sh, acc_sh,
               sa_v, da_v, sb_v, db_v, rows_v, gsem, isem):
    def fire(g, buf):
        s_v, d_v = buf
        pltpu.async_copy(src2d.at[pl.ds(base + g * 4, 4)], s_v, isem)
        pltpu.async_copy(dst2d.at[pl.ds(base + g * 4, 4)], d_v, isem)

    def drain(buf):
        s_v, d_v = buf
        pltpu.make_async_copy(src2d.at[pl.ds(base, 4)], s_v, isem).wait()
        pltpu.make_async_copy(dst2d.at[pl.ds(base, 4)], d_v, isem).wait()

    def process(buf):
        s_v, d_v = buf
        cps = [pltpu.async_copy(tab_sh.at[s_v.at[j]],
                                rows_v.at[pl.ds(j * 128, 128)], gsem)
               for j in range(4)]
        for cp in cps:
            cp.wait()
        for j in range(4):
            pltpu.sync_copy(rows_v.at[pl.ds(j * 128, 128)],
                            acc_sh.at[d_v.at[j]], add=True)

    _pipe4(n_groups, fire, drain, process, (sa_v, da_v), (sb_v, db_v))


@functools.partial(
    pl.kernel,
    out_type=[jax.ShapeDtypeStruct((NPAD, H), jnp.float32),
              jax.ShapeDtypeStruct((NC * NPAD,), jnp.float32)],
    mesh=_mesh,
    scratch_types=[
        pltpu.VMEM_SHARED((NPAD, 16), jnp.float32),
        pltpu.VMEM_SHARED((NPAD, 16), jnp.float32),
        pltpu.VMEM_SHARED((NPAD,), jnp.float32),
        pltpu.VMEM((ZR, 16), jnp.float32),
        pltpu.VMEM((APT,), jnp.float32),
        pltpu.VMEM((4, 128), jnp.int32),
        pltpu.VMEM((4, 128), jnp.int32),
        pltpu.VMEM((4, 128), jnp.int32),
        pltpu.VMEM((4, 128), jnp.int32),
        pltpu.VMEM((512, 16), jnp.float32),
        pltpu.VMEM((128,), jnp.float32),
        pltpu.SemaphoreType.DMA,
        pltpu.SemaphoreType.DMA,
    ],
    compiler_params=_sc_params,
)
def _agg1_kernel(y1, src2d, dst2d, agg_out, cnt_out,
                 tab_sh, acc_sh, cnt_sh, zb_v, zb1_v,
                 sa_v, da_v, sb_v, db_v, rows_v, ones_v, gsem, isem):
    c = lax.axis_index("c")
    s = lax.axis_index("s")
    for k in range(8):
        ones_v[pl.ds(k * 16, 16)] = jnp.full((16,), 1.0, jnp.float32)
    _fill_zero(zb_v)

    def zb1_body(r, carry):
        zb1_v[pl.ds(r * 16, 16)] = jnp.zeros((16,), jnp.float32)
        return carry
    lax.fori_loop(0, APT // 16, zb1_body, 0)
    pltpu.sync_copy(zb1_v, cnt_sh.at[pl.ds(s * APT, APT)])

    for q in range(2):
        qi = c * 2 + q
        pltpu.sync_copy(y1.at[pl.ds(s * APT, APT), pl.ds(16 * qi, 16)],
                        tab_sh.at[pl.ds(s * APT, APT)])
        _zero_shared_stripe(zb_v, acc_sh, s)
        plsc.subcore_barrier()

        _edge_loop(RPT // 4, s * RPT, src2d, dst2d, tab_sh, acc_sh,
                   sa_v, da_v, sb_v, db_v, rows_v, gsem, isem)

        if q == 0:
            cbase = c * RPC + s * RPCT

            def cfire(g, d_v):
                pltpu.async_copy(dst2d.at[pl.ds(cbase + g * 4, 4)],
                                 d_v, isem)

            def cdrain(d_v):
                pltpu.make_async_copy(dst2d.at[pl.ds(cbase, 4)],
                                      d_v, isem).wait()

            def cproc(d_v):
                for j in range(4):
                    pltpu.sync_copy(ones_v, cnt_sh.at[d_v.at[j]], add=True)

            _pipe4(RPCT // 4, cfire, cdrain, cproc, da_v, db_v)

        plsc.subcore_barrier()
        pltpu.sync_copy(acc_sh.at[pl.ds(s * APT, APT)],
                        agg_out.at[pl.ds(s * APT, APT), pl.ds(16 * qi, 16)])
        if q == 0:
            pltpu.sync_copy(cnt_sh.at[pl.ds(s * APT, APT)],
                            cnt_out.at[pl.ds(c * NPAD + s * APT, APT)])
        plsc.subcore_barrier()


@functools.partial(
    pl.kernel,
    out_type=jax.ShapeDtypeStruct((NPAD, 32), jnp.float32),
    mesh=_mesh,
    scratch_types=[
        pltpu.VMEM_SHARED((NPAD, 16), jnp.float32),
        pltpu.VMEM_SHARED((NPAD, 16), jnp.float32),
        pltpu.VMEM((ZR, 16), jnp.float32),
        pltpu.VMEM((4, 128), jnp.int32),
        pltpu.VMEM((4, 128), jnp.int32),
        pltpu.VMEM((4, 128), jnp.int32),
        pltpu.VMEM((4, 128), jnp.int32),
        pltpu.VMEM((512, 16), jnp.float32),
        pltpu.SemaphoreType.DMA,
        pltpu.SemaphoreType.DMA,
    ],
    compiler_params=_sc_params,
)
def _agg2_kernel(h2pad, src2d, dst2d, agg_out,
                 tab_sh, acc_sh, zb_v, sa_v, da_v, sb_v, db_v,
                 rows_v, gsem, isem):
    c = lax.axis_index("c")
    s = lax.axis_index("s")
    pltpu.sync_copy(h2pad.at[pl.ds(s * APT, APT)],
                    tab_sh.at[pl.ds(s * APT, APT)])
    _fill_zero(zb_v)
    _zero_shared_stripe(zb_v, acc_sh, s)
    plsc.subcore_barrier()

    _edge_loop(RPCT // 4, c * RPC + s * RPCT, src2d, dst2d, tab_sh, acc_sh,
               sa_v, da_v, sb_v, db_v, rows_v, gsem, isem)

    plsc.subcore_barrier()
    pltpu.sync_copy(acc_sh.at[pl.ds(s * APT, APT)],
                    agg_out.at[pl.ds(s * APT, APT), pl.ds(16 * c, 16)])


def kernel(x, edge_index, W1l, b1l, W1r, W2l, b2l, W2r):
    src = edge_index[0].astype(jnp.int32)
    dst = edge_index[1].astype(jnp.int32)
    pad = EPAD - E
    psrc = (jnp.arange(pad, dtype=jnp.int32) * 7919) % N
    pdst = N + jnp.arange(pad, dtype=jnp.int32) % (NPAD - N)
    src2d = jnp.concatenate([src, psrc]).reshape(ROWS, 128)
    dst2d = jnp.concatenate([dst, pdst]).reshape(ROWS, 128)

    y1, z = pl.pallas_call(
        _mm1_body,
        grid=(GRID,),
        in_specs=[pl.BlockSpec((BM, H), lambda i: (i, 0)),
                  pl.BlockSpec((H, H), lambda i: (0, 0)),
                  pl.BlockSpec((H, H), lambda i: (0, 0))],
        out_specs=[pl.BlockSpec((BM, H), lambda i: (i, 0)),
                   pl.BlockSpec((BM, H), lambda i: (i, 0))],
        out_shape=[jax.ShapeDtypeStruct((NPAD, H), jnp.float32),
                   jax.ShapeDtypeStruct((NPAD, H), jnp.float32)],
    )(x, W1l, W1r)

    agg1, cnt_flat = _agg1_kernel(y1, src2d, dst2d)
    cnt = cnt_flat.reshape(NC, NPAD)

    W2lp = jnp.zeros((16, H), jnp.float32).at[:3].set(W2l)
    W2rp = jnp.zeros((16, H), jnp.float32).at[:3].set(W2r)
    b2p = jnp.zeros((1, 16), jnp.float32).at[0, :3].set(b2l)
    h2pad, hb = pl.pallas_call(
        _mid_body,
        grid=(GRID,),
        in_specs=[pl.BlockSpec((BM, H), lambda i: (i, 0)),
                  pl.BlockSpec((2, BM), lambda i: (0, i)),
                  pl.BlockSpec((BM, H), lambda i: (i, 0)),
                  pl.BlockSpec((1, H), lambda i: (0, 0)),
                  pl.BlockSpec((16, H), lambda i: (0, 0)),
                  pl.BlockSpec((16, H), lambda i: (0, 0)),
                  pl.BlockSpec((1, 16), lambda i: (0, 0))],
        out_specs=[pl.BlockSpec((BM, 16), lambda i: (i, 0)),
                   pl.BlockSpec((BM, 16), lambda i: (i, 0))],
        out_shape=[jax.ShapeDtypeStruct((NPAD, 16), jnp.float32),
                   jax.ShapeDtypeStruct((NPAD, 16), jnp.float32)],
    )(agg1, cnt, z, b1l.reshape(1, H), W2lp, W2rp, b2p)

    agg2 = _agg2_kernel(h2pad, src2d, dst2d)

    out16 = pl.pallas_call(
        _fin_body,
        grid=(GRID,),
        in_specs=[pl.BlockSpec((BM, 32), lambda i: (i, 0)),
                  pl.BlockSpec((2, BM), lambda i: (0, i)),
                  pl.BlockSpec((BM, 16), lambda i: (i, 0))],
        out_specs=pl.BlockSpec((BM, 16), lambda i: (i, 0)),
        out_shape=jax.ShapeDtypeStruct((NPAD, 16), jnp.float32),
    )(agg2, cnt, hb)
    return out16[:N, :3]


# speedup vs baseline: 11.6505x; 1.0452x over previous
import functools

import jax
import jax.numpy as jnp
from jax import lax
from jax.experimental import pallas as pl
from jax.experimental.pallas import tpu as pltpu
from jax.experimental.pallas import tpu_sc as plsc

N = 50000
E = 800000
H = 64
NC, NS = 2, 16
EPAD = 802816
ROWS = EPAD // 128
RPT = ROWS // NS
RPC = ROWS // NC
RPCT = RPC // NS
NPAD = 51200
APT = NPAD // NS
BM = 2048
GRID = NPAD // BM
ZR = 400

_mesh = plsc.VectorSubcoreMesh(core_axis_name="c", subcore_axis_name="s")
_sc_params = pltpu.CompilerParams(use_tc_tiling_on_sc=False)


def _mm1_body(x_ref, wl_ref, wr_ref, y1_ref, z_ref):
    xb = x_ref[...]
    y1_ref[...] = lax.dot_general(xb, wl_ref[...], (((1,), (1,)), ((), ())),
                                  preferred_element_type=jnp.float32)
    z_ref[...] = lax.dot_general(xb, wr_ref[...], (((1,), (1,)), ((), ())),
                                 preferred_element_type=jnp.float32)


def _mid_body(agg_ref, cnt_ref, z_ref, b1_ref, w2l_ref, w2r_ref, b2_ref,
              h2_ref, hb_ref):
    cnt = cnt_ref[0] + cnt_ref[1]
    d = 1.0 / jnp.maximum(cnt, 1.0)
    h = jnp.maximum(agg_ref[...] * d[:, None] + b1_ref[0][None, :]
                    + z_ref[...], 0.0)
    h2_ref[...] = lax.dot_general(h, w2l_ref[...], (((1,), (1,)), ((), ())),
                                  preferred_element_type=jnp.float32)
    hb_ref[...] = lax.dot_general(h, w2r_ref[...], (((1,), (1,)), ((), ())),
                                  preferred_element_type=jnp.float32) \
        + b2_ref[0][None, :]


def _fin_body(agg2_ref, cnt_ref, hb_ref, out_ref):
    a = agg2_ref[:, :16] + agg2_ref[:, 16:]
    cnt = cnt_ref[0] + cnt_ref[1]
    d = 1.0 / jnp.maximum(cnt, 1.0)
    out_ref[...] = a * d[:, None] + hb_ref[...]


def _fill_zero(zb_v):
    def body(r, carry):
        zb_v[r, :] = jnp.zeros((16,), jnp.float32)
        return carry
    lax.fori_loop(0, ZR, body, 0)


def _zero_shared_stripe(zb_v, buf_sh, s):
    for k in range(APT // ZR):
        pltpu.sync_copy(zb_v, buf_sh.at[pl.ds(s * APT + k * ZR, ZR)])


def _pipe4(n_groups, fire, drain, process, bufa, bufb):
    npairs, tail = n_groups // 2, n_groups % 2
    fire(0, bufa)

    def body(p, carry):
        g = p * 2
        fire(g + 1, bufb)
        drain(bufa)
        process(bufa)
        fire(jnp.minimum(g + 2, n_groups - 1), bufa)
        drain(bufb)
        process(bufb)
        return carry
    lax.fori_loop(0, npairs, body, 0)
    drain(bufa)
    if tail:
        process(bufa)


def _edge_loop(n_groups, base, src2d, dst2d, tab_sh, acc_sh,
               sa_v, da_v, sb_v, db_v, rows_v, gsem, isem):
    def fire(g, buf):
        s_v, d_v = buf
        pltpu.async_copy(src2d.at[pl.ds(base + g * 4, 4)], s_v, isem)
        pltpu.async_copy(dst2d.at[pl.ds(base + g * 4, 4)], d_v, isem)

    def drain(buf):
        s_v, d_v = buf
        pltpu.make_async_copy(src2d.at[pl.ds(base, 4)], s_v, isem).wait()
        pltpu.make_async_copy(dst2d.at[pl.ds(base, 4)], d_v, isem).wait()

    def process(buf):
        s_v, d_v = buf
        cps = [pltpu.async_copy(tab_sh.at[s_v.at[j]],
                                rows_v.at[pl.ds(j * 128, 128)], gsem)
               for j in range(4)]
        for cp in cps:
            cp.wait()
        scs = [pltpu.async_copy(rows_v.at[pl.ds(j * 128, 128)],
                                acc_sh.at[d_v.at[j]], gsem, add=True)
               for j in range(4)]
        for sc in scs:
            sc.wait()

    _pipe4(n_groups, fire, drain, process, (sa_v, da_v), (sb_v, db_v))


@functools.partial(
    pl.kernel,
    out_type=[jax.ShapeDtypeStruct((NPAD, H), jnp.float32),
              jax.ShapeDtypeStruct((NC * NPAD,), jnp.float32)],
    mesh=_mesh,
    scratch_types=[
        pltpu.VMEM_SHARED((NPAD, 16), jnp.float32),
        pltpu.VMEM_SHARED((NPAD, 16), jnp.float32),
        pltpu.VMEM_SHARED((NPAD,), jnp.float32),
        pltpu.VMEM((ZR, 16), jnp.float32),
        pltpu.VMEM((APT,), jnp.float32),
        pltpu.VMEM((4, 128), jnp.int32),
        pltpu.VMEM((4, 128), jnp.int32),
        pltpu.VMEM((4, 128), jnp.int32),
        pltpu.VMEM((4, 128), jnp.int32),
        pltpu.VMEM((512, 16), jnp.float32),
        pltpu.VMEM((128,), jnp.float32),
        pltpu.SemaphoreType.DMA,
        pltpu.SemaphoreType.DMA,
    ],
    compiler_params=_sc_params,
)
def _agg1_kernel(y1, src2d, dst2d, agg_out, cnt_out,
                 tab_sh, acc_sh, cnt_sh, zb_v, zb1_v,
                 sa_v, da_v, sb_v, db_v, rows_v, ones_v, gsem, isem):
    c = lax.axis_index("c")
    s = lax.axis_index("s")
    for k in range(8):
        ones_v[pl.ds(k * 16, 16)] = jnp.full((16,), 1.0, jnp.float32)
    _fill_zero(zb_v)

    def zb1_body(r, carry):
        zb1_v[pl.ds(r * 16, 16)] = jnp.zeros((16,), jnp.float32)
        return carry
    lax.fori_loop(0, APT // 16, zb1_body, 0)
    pltpu.sync_copy(zb1_v, cnt_sh.at[pl.ds(s * APT, APT)])

    for q in range(2):
        qi = c * 2 + q
        pltpu.sync_copy(y1.at[pl.ds(s * APT, APT), pl.ds(16 * qi, 16)],
                        tab_sh.at[pl.ds(s * APT, APT)])
        _zero_shared_stripe(zb_v, acc_sh, s)
        plsc.subcore_barrier()

        _edge_loop(RPT // 4, s * RPT, src2d, dst2d, tab_sh, acc_sh,
                   sa_v, da_v, sb_v, db_v, rows_v, gsem, isem)

        if q == 0:
            cbase = c * RPC + s * RPCT

            def cfire(g, d_v):
                pltpu.async_copy(dst2d.at[pl.ds(cbase + g * 4, 4)],
                                 d_v, isem)

            def cdrain(d_v):
                pltpu.make_async_copy(dst2d.at[pl.ds(cbase, 4)],
                                      d_v, isem).wait()

            def cproc(d_v):
                scs = [pltpu.async_copy(ones_v, cnt_sh.at[d_v.at[j]],
                                        gsem, add=True)
                       for j in range(4)]
                for sc in scs:
                    sc.wait()

            _pipe4(RPCT // 4, cfire, cdrain, cproc, da_v, db_v)

        plsc.subcore_barrier()
        pltpu.sync_copy(acc_sh.at[pl.ds(s * APT, APT)],
                        agg_out.at[pl.ds(s * APT, APT), pl.ds(16 * qi, 16)])
        if q == 0:
            pltpu.sync_copy(cnt_sh.at[pl.ds(s * APT, APT)],
                            cnt_out.at[pl.ds(c * NPAD + s * APT, APT)])
        plsc.subcore_barrier()


@functools.partial(
    pl.kernel,
    out_type=jax.ShapeDtypeStruct((NPAD, 32), jnp.float32),
    mesh=_mesh,
    scratch_types=[
        pltpu.VMEM_SHARED((NPAD, 16), jnp.float32),
        pltpu.VMEM_SHARED((NPAD, 16), jnp.float32),
        pltpu.VMEM((ZR, 16), jnp.float32),
        pltpu.VMEM((4, 128), jnp.int32),
        pltpu.VMEM((4, 128), jnp.int32),
        pltpu.VMEM((4, 128), jnp.int32),
        pltpu.VMEM((4, 128), jnp.int32),
        pltpu.VMEM((512, 16), jnp.float32),
        pltpu.SemaphoreType.DMA,
        pltpu.SemaphoreType.DMA,
    ],
    compiler_params=_sc_params,
)
def _agg2_kernel(h2pad, src2d, dst2d, agg_out,
                 tab_sh, acc_sh, zb_v, sa_v, da_v, sb_v, db_v,
                 rows_v, gsem, isem):
    c = lax.axis_index("c")
    s = lax.axis_index("s")
    pltpu.sync_copy(h2pad.at[pl.ds(s * APT, APT)],
                    tab_sh.at[pl.ds(s * APT, APT)])
    _fill_zero(zb_v)
    _zero_shared_stripe(zb_v, acc_sh, s)
    plsc.subcore_barrier()

    _edge_loop(RPCT // 4, c * RPC + s * RPCT, src2d, dst2d, tab_sh, acc_sh,
               sa_v, da_v, sb_v, db_v, rows_v, gsem, isem)

    plsc.subcore_barrier()
    pltpu.sync_copy(acc_sh.at[pl.ds(s * APT, APT)],
                    agg_out.at[pl.ds(s * APT, APT), pl.ds(16 * c, 16)])


def kernel(x, edge_index, W1l, b1l, W1r, W2l, b2l, W2r):
    src = edge_index[0].astype(jnp.int32)
    dst = edge_index[1].astype(jnp.int32)
    pad = EPAD - E
    psrc = (jnp.arange(pad, dtype=jnp.int32) * 7919) % N
    pdst = N + jnp.arange(pad, dtype=jnp.int32) % (NPAD - N)
    src2d = jnp.concatenate([src, psrc]).reshape(ROWS, 128)
    dst2d = jnp.concatenate([dst, pdst]).reshape(ROWS, 128)

    y1, z = pl.pallas_call(
        _mm1_body,
        grid=(GRID,),
        in_specs=[pl.BlockSpec((BM, H), lambda i: (i, 0)),
                  pl.BlockSpec((H, H), lambda i: (0, 0)),
                  pl.BlockSpec((H, H), lambda i: (0, 0))],
        out_specs=[pl.BlockSpec((BM, H), lambda i: (i, 0)),
                   pl.BlockSpec((BM, H), lambda i: (i, 0))],
        out_shape=[jax.ShapeDtypeStruct((NPAD, H), jnp.float32),
                   jax.ShapeDtypeStruct((NPAD, H), jnp.float32)],
    )(x, W1l, W1r)

    agg1, cnt_flat = _agg1_kernel(y1, src2d, dst2d)
    cnt = cnt_flat.reshape(NC, NPAD)

    W2lp = jnp.zeros((16, H), jnp.float32).at[:3].set(W2l)
    W2rp = jnp.zeros((16, H), jnp.float32).at[:3].set(W2r)
    b2p = jnp.zeros((1, 16), jnp.float32).at[0, :3].set(b2l)
    h2pad, hb = pl.pallas_call(
        _mid_body,
        grid=(GRID,),
        in_specs=[pl.BlockSpec((BM, H), lambda i: (i, 0)),
                  pl.BlockSpec((2, BM), lambda i: (0, i)),
                  pl.BlockSpec((BM, H), lambda i: (i, 0)),
                  pl.BlockSpec((1, H), lambda i: (0, 0)),
                  pl.BlockSpec((16, H), lambda i: (0, 0)),
                  pl.BlockSpec((16, H), lambda i: (0, 0)),
                  pl.BlockSpec((1, 16), lambda i: (0, 0))],
        out_specs=[pl.BlockSpec((BM, 16), lambda i: (i, 0)),
                   pl.BlockSpec((BM, 16), lambda i: (i, 0))],
        out_shape=[jax.ShapeDtypeStruct((NPAD, 16), jnp.float32),
                   jax.ShapeDtypeStruct((NPAD, 16), jnp.float32)],
    )(agg1, cnt, z, b1l.reshape(1, H), W2lp, W2rp, b2p)

    agg2 = _agg2_kernel(h2pad, src2d, dst2d)

    out16 = pl.pallas_call(
        _fin_body,
        grid=(GRID,),
        in_specs=[pl.BlockSpec((BM, 32), lambda i: (i, 0)),
                  pl.BlockSpec((2, BM), lambda i: (0, i)),
                  pl.BlockSpec((BM, 16), lambda i: (i, 0))],
        out_specs=pl.BlockSpec((BM, 16), lambda i: (i, 0)),
        out_shape=jax.ShapeDtypeStruct((NPAD, 16), jnp.float32),
    )(agg2, cnt, hb)
    return out16[:N, :3]


# speedup vs baseline: 11.6703x; 1.0017x over previous
import functools

import jax
import jax.numpy as jnp
from jax import lax
from jax.experimental import pallas as pl
from jax.experimental.pallas import tpu as pltpu
from jax.experimental.pallas import tpu_sc as plsc

N = 50000
E = 800000
H = 64
NC, NS = 2, 16
EPAD = 802816
ROWS = EPAD // 128
RPT = ROWS // NS
RPC = ROWS // NC
RPCT = RPC // NS
NPAD = 51200
APT = NPAD // NS
BM = 2048
GRID = NPAD // BM
ZR = 400

_mesh = plsc.VectorSubcoreMesh(core_axis_name="c", subcore_axis_name="s")
_sc_params = pltpu.CompilerParams(use_tc_tiling_on_sc=False)


def _mm1_body(x_ref, wl_ref, wr_ref, y1_ref, z_ref):
    xb = x_ref[...]
    y1_ref[...] = lax.dot_general(xb, wl_ref[...], (((1,), (1,)), ((), ())),
                                  preferred_element_type=jnp.float32)
    z_ref[...] = lax.dot_general(xb, wr_ref[...], (((1,), (1,)), ((), ())),
                                 preferred_element_type=jnp.float32)


def _mid_body(agg_ref, cnt_ref, z_ref, b1_ref, w2l_ref, w2r_ref, b2_ref,
              h2_ref, hb_ref):
    cnt = cnt_ref[0] + cnt_ref[1]
    d = 1.0 / jnp.maximum(cnt, 1.0)
    h = jnp.maximum(agg_ref[...] * d[:, None] + b1_ref[0][None, :]
                    + z_ref[...], 0.0)
    h2_ref[...] = lax.dot_general(h, w2l_ref[...], (((1,), (1,)), ((), ())),
                                  preferred_element_type=jnp.float32)
    hb_ref[...] = lax.dot_general(h, w2r_ref[...], (((1,), (1,)), ((), ())),
                                  preferred_element_type=jnp.float32) \
        + b2_ref[0][None, :]


def _fin_body(agg2_ref, cnt_ref, hb_ref, out_ref):
    a = agg2_ref[:, :16] + agg2_ref[:, 16:]
    cnt = cnt_ref[0] + cnt_ref[1]
    d = 1.0 / jnp.maximum(cnt, 1.0)
    out_ref[:, :16] = a * d[:, None] + hb_ref[...]


def _fill_zero(zb_v):
    def body(r, carry):
        zb_v[r, :] = jnp.zeros((16,), jnp.float32)
        return carry
    lax.fori_loop(0, ZR, body, 0)


def _zero_shared_stripe(zb_v, buf_sh, s):
    for k in range(APT // ZR):
        pltpu.sync_copy(zb_v, buf_sh.at[pl.ds(s * APT + k * ZR, ZR)])


def _pipe4(n_groups, fire, drain, process, bufa, bufb):
    npairs, tail = n_groups // 2, n_groups % 2
    fire(0, bufa)

    def body(p, carry):
        g = p * 2
        fire(g + 1, bufb)
        drain(bufa)
        process(bufa)
        fire(jnp.minimum(g + 2, n_groups - 1), bufa)
        drain(bufb)
        process(bufb)
        return carry
    lax.fori_loop(0, npairs, body, 0)
    drain(bufa)
    if tail:
        process(bufa)


def _edge_loop(n_groups, base, src2d, dst2d, tab_sh, acc_sh,
               sa_v, da_v, sb_v, db_v, rows_v, gsem, isem):
    def fire(g, buf):
        s_v, d_v = buf
        pltpu.async_copy(src2d.at[pl.ds(base + g * 4, 4)], s_v, isem)
        pltpu.async_copy(dst2d.at[pl.ds(base + g * 4, 4)], d_v, isem)

    def drain(buf):
        s_v, d_v = buf
        pltpu.make_async_copy(src2d.at[pl.ds(base, 4)], s_v, isem).wait()
        pltpu.make_async_copy(dst2d.at[pl.ds(base, 4)], d_v, isem).wait()

    def process(buf):
        s_v, d_v = buf
        cps = [pltpu.async_copy(tab_sh.at[s_v.at[j]],
                                rows_v.at[pl.ds(j * 128, 128)], gsem)
               for j in range(4)]
        for cp in cps:
            cp.wait()
        scs = [pltpu.async_copy(rows_v.at[pl.ds(j * 128, 128)],
                                acc_sh.at[d_v.at[j]], gsem, add=True)
               for j in range(4)]
        for sc in scs:
            sc.wait()

    _pipe4(n_groups, fire, drain, process, (sa_v, da_v), (sb_v, db_v))


@functools.partial(
    pl.kernel,
    out_type=[jax.ShapeDtypeStruct((NPAD, H), jnp.float32),
              jax.ShapeDtypeStruct((NC * NPAD,), jnp.float32)],
    mesh=_mesh,
    scratch_types=[
        pltpu.VMEM_SHARED((NPAD, 16), jnp.float32),
        pltpu.VMEM_SHARED((NPAD, 16), jnp.float32),
        pltpu.VMEM_SHARED((NPAD,), jnp.float32),
        pltpu.VMEM((ZR, 16), jnp.float32),
        pltpu.VMEM((APT,), jnp.float32),
        pltpu.VMEM((4, 128), jnp.int32),
        pltpu.VMEM((4, 128), jnp.int32),
        pltpu.VMEM((4, 128), jnp.int32),
        pltpu.VMEM((4, 128), jnp.int32),
        pltpu.VMEM((512, 16), jnp.float32),
        pltpu.VMEM((128,), jnp.float32),
        pltpu.SemaphoreType.DMA,
        pltpu.SemaphoreType.DMA,
    ],
    compiler_params=_sc_params,
)
def _agg1_kernel(y1, src2d, dst2d, agg_out, cnt_out,
                 tab_sh, acc_sh, cnt_sh, zb_v, zb1_v,
                 sa_v, da_v, sb_v, db_v, rows_v, ones_v, gsem, isem):
    c = lax.axis_index("c")
    s = lax.axis_index("s")
    for k in range(8):
        ones_v[pl.ds(k * 16, 16)] = jnp.full((16,), 1.0, jnp.float32)
    _fill_zero(zb_v)

    def zb1_body(r, carry):
        zb1_v[pl.ds(r * 16, 16)] = jnp.zeros((16,), jnp.float32)
        return carry
    lax.fori_loop(0, APT // 16, zb1_body, 0)
    pltpu.sync_copy(zb1_v, cnt_sh.at[pl.ds(s * APT, APT)])

    for q in range(2):
        qi = c * 2 + q
        pltpu.sync_copy(y1.at[pl.ds(s * APT, APT), pl.ds(16 * qi, 16)],
                        tab_sh.at[pl.ds(s * APT, APT)])
        _zero_shared_stripe(zb_v, acc_sh, s)
        plsc.subcore_barrier()

        _edge_loop(RPT // 4, s * RPT, src2d, dst2d, tab_sh, acc_sh,
                   sa_v, da_v, sb_v, db_v, rows_v, gsem, isem)

        if q == 0:
            cbase = c * RPC + s * RPCT

            def cfire(g, d_v):
                pltpu.async_copy(dst2d.at[pl.ds(cbase + g * 4, 4)],
                                 d_v, isem)

            def cdrain(d_v):
                pltpu.make_async_copy(dst2d.at[pl.ds(cbase, 4)],
                                      d_v, isem).wait()

            def cproc(d_v):
                scs = [pltpu.async_copy(ones_v, cnt_sh.at[d_v.at[j]],
                                        gsem, add=True)
                       for j in range(4)]
                for sc in scs:
                    sc.wait()

            _pipe4(RPCT // 4, cfire, cdrain, cproc, da_v, db_v)

        plsc.subcore_barrier()
        pltpu.sync_copy(acc_sh.at[pl.ds(s * APT, APT)],
                        agg_out.at[pl.ds(s * APT, APT), pl.ds(16 * qi, 16)])
        if q == 0:
            pltpu.sync_copy(cnt_sh.at[pl.ds(s * APT, APT)],
                            cnt_out.at[pl.ds(c * NPAD + s * APT, APT)])
        plsc.subcore_barrier()


@functools.partial(
    pl.kernel,
    out_type=jax.ShapeDtypeStruct((NPAD, 32), jnp.float32),
    mesh=_mesh,
    scratch_types=[
        pltpu.VMEM_SHARED((NPAD, 16), jnp.float32),
        pltpu.VMEM_SHARED((NPAD, 16), jnp.float32),
        pltpu.VMEM((ZR, 16), jnp.float32),
        pltpu.VMEM((4, 128), jnp.int32),
        pltpu.VMEM((4, 128), jnp.int32),
        pltpu.VMEM((4, 128), jnp.int32),
        pltpu.VMEM((4, 128), jnp.int32),
        pltpu.VMEM((512, 16), jnp.float32),
        pltpu.SemaphoreType.DMA,
        pltpu.SemaphoreType.DMA,
    ],
    compiler_params=_sc_params,
)
def _agg2_kernel(h2pad, src2d, dst2d, agg_out,
                 tab_sh, acc_sh, zb_v, sa_v, da_v, sb_v, db_v,
                 rows_v, gsem, isem):
    c = lax.axis_index("c")
    s = lax.axis_index("s")
    pltpu.sync_copy(h2pad.at[pl.ds(s * APT, APT)],
                    tab_sh.at[pl.ds(s * APT, APT)])
    _fill_zero(zb_v)
    _zero_shared_stripe(zb_v, acc_sh, s)
    plsc.subcore_barrier()

    _edge_loop(RPCT // 4, c * RPC + s * RPCT, src2d, dst2d, tab_sh, acc_sh,
               sa_v, da_v, sb_v, db_v, rows_v, gsem, isem)

    plsc.subcore_barrier()
    pltpu.sync_copy(acc_sh.at[pl.ds(s * APT, APT)],
                    agg_out.at[pl.ds(s * APT, APT), pl.ds(16 * c, 16)])


def kernel(x, edge_index, W1l, b1l, W1r, W2l, b2l, W2r):
    src = edge_index[0].astype(jnp.int32)
    dst = edge_index[1].astype(jnp.int32)
    pad = EPAD - E
    psrc = (jnp.arange(pad, dtype=jnp.int32) * 7919) % N
    pdst = N + jnp.arange(pad, dtype=jnp.int32) % (NPAD - N)
    src2d = jnp.concatenate([src, psrc]).reshape(ROWS, 128)
    dst2d = jnp.concatenate([dst, pdst]).reshape(ROWS, 128)

    y1, z = pl.pallas_call(
        _mm1_body,
        grid=(GRID,),
        in_specs=[pl.BlockSpec((BM, H), lambda i: (i, 0)),
                  pl.BlockSpec((H, H), lambda i: (0, 0)),
                  pl.BlockSpec((H, H), lambda i: (0, 0))],
        out_specs=[pl.BlockSpec((BM, H), lambda i: (i, 0)),
                   pl.BlockSpec((BM, H), lambda i: (i, 0))],
        out_shape=[jax.ShapeDtypeStruct((NPAD, H), jnp.float32),
                   jax.ShapeDtypeStruct((NPAD, H), jnp.float32)],
    )(x, W1l, W1r)

    agg1, cnt_flat = _agg1_kernel(y1, src2d, dst2d)
    cnt = cnt_flat.reshape(NC, NPAD)

    W2lp = jnp.zeros((16, H), jnp.float32).at[:3].set(W2l)
    W2rp = jnp.zeros((16, H), jnp.float32).at[:3].set(W2r)
    b2p = jnp.zeros((1, 16), jnp.float32).at[0, :3].set(b2l)
    h2pad, hb = pl.pallas_call(
        _mid_body,
        grid=(GRID,),
        in_specs=[pl.BlockSpec((BM, H), lambda i: (i, 0)),
                  pl.BlockSpec((2, BM), lambda i: (0, i)),
                  pl.BlockSpec((BM, H), lambda i: (i, 0)),
                  pl.BlockSpec((1, H), lambda i: (0, 0)),
                  pl.BlockSpec((16, H), lambda i: (0, 0)),
                  pl.BlockSpec((16, H), lambda i: (0, 0)),
                  pl.BlockSpec((1, 16), lambda i: (0, 0))],
        out_specs=[pl.BlockSpec((BM, 16), lambda i: (i, 0)),
                   pl.BlockSpec((BM, 16), lambda i: (i, 0))],
        out_shape=[jax.ShapeDtypeStruct((NPAD, 16), jnp.float32),
                   jax.ShapeDtypeStruct((NPAD, 16), jnp.float32)],
    )(agg1, cnt, z, b1l.reshape(1, H), W2lp, W2rp, b2p)

    agg2 = _agg2_kernel(h2pad, src2d, dst2d)

    out16 = pl.pallas_call(
        _fin_body,
        grid=(GRID,),
        in_specs=[pl.BlockSpec((BM, 32), lambda i: (i, 0)),
                  pl.BlockSpec((2, BM), lambda i: (0, i)),
                  pl.BlockSpec((BM, 16), lambda i: (i, 0))],
        out_specs=pl.BlockSpec((BM, 128), lambda i: (i, 0)),
        out_shape=jax.ShapeDtypeStruct((NPAD, 128), jnp.float32),
    )(agg2, cnt, hb)
    return out16[:N, :3]
